# baseline (speedup 1.0000x reference)
import functools

import jax
import jax.numpy as jnp
from jax import lax
from jax.experimental import pallas as pl
from jax.experimental.pallas import tpu as pltpu
from jax.experimental.pallas import tpu_sc as plsc

N_NODES = 10000
D = 128
N_GRAPHS = 16
E = 320000

NC, NS = 2, 16
NW = NC * NS
CHUNK = 96
GRP = 27
NGRP0 = 8
CHUNKS0 = GRP * NGRP0
E_PAD = NS * CHUNKS0 * CHUNK
DEG_E_PER_TILE = E_PAD // NW
ACC_ROWS = 10240
ROWS_PER_TILE = ACC_ROWS // NS

NODE_BLK = 1024
N_BLKS = ACC_ROWS // NODE_BLK


def _sc_agg_body(y_hbm, eidx_hbm, acc_out,
                 eidx_v, rows0, rows1, rows2, acc_sh,
                 gsem0, gsem1, gsem2, ssem0, ssem1, ssem2):
    cid = lax.axis_index("c")
    sid = lax.axis_index("s")

    def _zrow(r, carry):
        for j in range(D // 16):
            rows0[r, pl.ds(j * 16, 16)] = jnp.zeros((16,), jnp.float32)
        return carry
    lax.fori_loop(0, CHUNK, _zrow, 0)
    for j in range(ROWS_PER_TILE // CHUNK):
        pltpu.sync_copy(
            rows0, acc_sh.at[pl.ds(sid * ROWS_PER_TILE + j * CHUNK, CHUNK)])
    rem = ROWS_PER_TILE % CHUNK
    if rem:
        base = sid * ROWS_PER_TILE + (ROWS_PER_TILE // CHUNK) * CHUNK
        pltpu.sync_copy(rows0.at[pl.ds(0, rem)], acc_sh.at[pl.ds(base, rem)])

    rows = (rows0, rows1, rows2)
    gsem = (gsem0, gsem1, gsem2)
    ssem = (ssem0, ssem1, ssem2)

    cbase = sid * CHUNKS0

    def _run_group(first_row):
        pltpu.sync_copy(eidx_hbm.at[pl.ds(first_row, GRP)], eidx_v)
        pltpu.async_copy(y_hbm.at[eidx_v.at[0, 0]], rows0, gsem0)
        pltpu.async_copy(y_hbm.at[eidx_v.at[1, 0]], rows1, gsem1)

        def _triple(i, carry):
            for b in range(3):
                cc = i * 3 + b
                pltpu.make_async_copy(
                    y_hbm.at[eidx_v.at[cc, 0]], rows[b], gsem[b]).wait()
                pltpu.async_copy(
                    rows[b], acc_sh.at[eidx_v.at[cc, 1]], ssem[b], add=True)
                pb = (b - 1) % 3

                @pl.when(cc >= 1)
                def _():
                    pltpu.make_async_copy(
                        rows[pb], acc_sh.at[eidx_v.at[cc, 1]],
                        ssem[pb]).wait()

                @pl.when(cc + 2 < GRP)
                def _():
                    pltpu.async_copy(
                        y_hbm.at[eidx_v.at[cc + 2, 0]], rows[pb], gsem[pb])
            return carry
        lax.fori_loop(0, GRP // 3, _triple, 0)
        pltpu.make_async_copy(
            rows[(GRP - 1) % 3], acc_sh.at[eidx_v.at[GRP - 1, 1]],
            ssem[(GRP - 1) % 3]).wait()

    plsc.subcore_barrier()
    for q in range(NGRP0):
        @pl.when(cid == 0)
        def _(q=q):
            _run_group(cbase + q * GRP)

    plsc.subcore_barrier()

    @pl.when(cid == 0)
    def _():
        pltpu.sync_copy(
            acc_sh.at[pl.ds(sid * ROWS_PER_TILE, ROWS_PER_TILE)],
            acc_out.at[pl.ds(sid * ROWS_PER_TILE, ROWS_PER_TILE)])


def _sc_deg_body(dst_hbm, deg_out, didx_v, deg_v):
    cid = lax.axis_index("c")
    sid = lax.axis_index("s")
    wid = cid * NS + sid

    pltpu.sync_copy(
        dst_hbm.at[pl.ds(wid * DEG_E_PER_TILE, DEG_E_PER_TILE)], didx_v)

    def _zdeg(i, carry):
        deg_v[pl.ds(i * 16, 16)] = jnp.zeros((16,), jnp.float32)
        return carry
    lax.fori_loop(0, ACC_ROWS // 16, _zdeg, 0)

    ones = jnp.ones((16,), jnp.float32)

    def _hist(i, carry):
        iv = didx_v[pl.ds(i * 16, 16)]
        plsc.addupdate_scatter(deg_v, [iv], ones)
        return carry
    lax.fori_loop(0, DEG_E_PER_TILE // 16, _hist, 0)

    pltpu.sync_copy(deg_v, deg_out.at[wid])


@functools.cache
def _get_sc_agg():
    return functools.partial(
        pl.kernel,
        mesh=plsc.VectorSubcoreMesh(core_axis_name="c", subcore_axis_name="s"),
        compiler_params=pltpu.CompilerParams(needs_layout_passes=False),
        out_type=jax.ShapeDtypeStruct((ACC_ROWS, D), jnp.float32),
        scratch_types=[
            pltpu.VMEM((GRP, 2, CHUNK), jnp.int32),
            pltpu.VMEM((CHUNK, D), jnp.float32),
            pltpu.VMEM((CHUNK, D), jnp.float32),
            pltpu.VMEM((CHUNK, D), jnp.float32),
            pltpu.VMEM_SHARED((ACC_ROWS, D), jnp.float32),
            pltpu.SemaphoreType.DMA,
            pltpu.SemaphoreType.DMA,
            pltpu.SemaphoreType.DMA,
            pltpu.SemaphoreType.DMA,
            pltpu.SemaphoreType.DMA,
            pltpu.SemaphoreType.DMA,
        ],
    )(_sc_agg_body)


@functools.cache
def _get_sc_deg():
    return functools.partial(
        pl.kernel,
        mesh=plsc.VectorSubcoreMesh(core_axis_name="c", subcore_axis_name="s"),
        compiler_params=pltpu.CompilerParams(needs_layout_passes=False),
        out_type=jax.ShapeDtypeStruct((NW, ACC_ROWS), jnp.float32),
        scratch_types=[
            pltpu.VMEM((DEG_E_PER_TILE,), jnp.int32),
            pltpu.VMEM((ACC_ROWS,), jnp.float32),
        ],
    )(_sc_deg_body)


def _tc1_body(x_ref, wa_ref, wb_ref, y_ref, r_ref):
    xb = x_ref[...]
    y_ref[...] = jnp.dot(xb, wa_ref[...], preferred_element_type=jnp.float32)
    r_ref[...] = jnp.dot(xb, wb_ref[...], preferred_element_type=jnp.float32)


def _conv_epilogue(acc_ref, deg_ref, r_ref, b_ref):
    deg = jnp.sum(deg_ref[...], axis=0)
    mean = acc_ref[...] / jnp.maximum(deg, 1.0)[:, None]
    return jnp.maximum(mean + b_ref[...] + r_ref[...], 0.0)


def _pool_contrib(batch_ref, h):
    b = batch_ref[0, 0, :]
    g = lax.broadcasted_iota(jnp.int32, (N_GRAPHS, NODE_BLK), 0)
    onehot = (g == b[None, :]).astype(jnp.float32)
    return jnp.dot(onehot, h, preferred_element_type=jnp.float32)


def _tc2_body(acc_ref, deg_ref, r_ref, b_ref, batch_ref, wa_ref, wb_ref,
              y2_ref, r2_ref, pool_ref):
    i = pl.program_id(0)
    h = _conv_epilogue(acc_ref, deg_ref, r_ref, b_ref)
    y2_ref[...] = jnp.dot(h, wa_ref[...], preferred_element_type=jnp.float32)
    r2_ref[...] = jnp.dot(h, wb_ref[...], preferred_element_type=jnp.float32)

    @pl.when(i == 0)
    def _():
        pool_ref[...] = jnp.zeros_like(pool_ref)
    pool_ref[...] += _pool_contrib(batch_ref, h)


def _tc3_body(acc_ref, deg_ref, r_ref, b_ref, batch_ref, pool1_ref,
              w1_ref, b1_ref, w2_ref, b2_ref, w3_ref, b3_ref,
              out_ref, pool2_ref):
    i = pl.program_id(0)
    h = _conv_epilogue(acc_ref, deg_ref, r_ref, b_ref)

    @pl.when(i == 0)
    def _():
        pool2_ref[...] = jnp.zeros_like(pool2_ref)
    pool2_ref[...] += _pool_contrib(batch_ref, h)

    @pl.when(i == N_BLKS - 1)
    def _():
        z = jnp.concatenate([pool1_ref[...], pool2_ref[...]], axis=-1)
        z = jnp.maximum(
            jnp.dot(z, w1_ref[...], preferred_element_type=jnp.float32)
            + b1_ref[...], 0.0)
        z = jnp.maximum(
            jnp.dot(z, w2_ref[...], preferred_element_type=jnp.float32)
            + b2_ref[...], 0.0)
        z = (jnp.dot(z, w3_ref[...], preferred_element_type=jnp.float32)
             + b3_ref[...])
        m = jnp.max(z, axis=-1, keepdims=True)
        lse = jnp.log(jnp.sum(jnp.exp(z - m), axis=-1, keepdims=True)) + m
        out_ref[...] = z - lse


def _full_spec(shape):
    return pl.BlockSpec(shape, lambda i: tuple(0 for _ in shape))


def kernel(x, edge_index, batch, W_rel1, b_rel1, W_root1, W_rel2, b_rel2,
           W_root2, W1, b1, W2, b2, W3, b3):
    src, dst = edge_index[0], edge_index[1]
    pad = E_PAD - E
    src_p = jnp.concatenate([src, jnp.zeros((pad,), jnp.int32)])
    pad_dst = N_NODES + jnp.arange(pad, dtype=jnp.int32) % (ACC_ROWS - N_NODES)
    dst_p = jnp.concatenate([dst, pad_dst])
    eidx = jnp.stack(
        [src_p.reshape(-1, CHUNK), dst_p.reshape(-1, CHUNK)], axis=1)
    x_p = jnp.concatenate([x, jnp.zeros((ACC_ROWS - N_NODES, D), jnp.float32)])
    batch_p = jnp.concatenate(
        [batch, jnp.full((ACC_ROWS - N_NODES,), N_GRAPHS, jnp.int32)])
    batch3 = batch_p.reshape(N_BLKS, 1, NODE_BLK)

    y1, r1 = pl.pallas_call(
        _tc1_body,
        grid=(N_BLKS,),
        in_specs=[
            pl.BlockSpec((NODE_BLK, D), lambda i: (i, 0)),
            _full_spec((D, D)),
            _full_spec((D, D)),
        ],
        out_specs=[pl.BlockSpec((NODE_BLK, D), lambda i: (i, 0))] * 2,
        out_shape=[jax.ShapeDtypeStruct((ACC_ROWS, D), jnp.float32)] * 2,
    )(x_p, W_rel1, W_root1)

    degp = _get_sc_deg()(dst_p)
    acc1 = _get_sc_agg()(y1, eidx)

    blk_spec = pl.BlockSpec((NODE_BLK, D), lambda i: (i, 0))
    acc_spec = blk_spec
    deg_spec = pl.BlockSpec((NW, NODE_BLK), lambda i: (0, i))
    batch_spec = pl.BlockSpec((1, 1, NODE_BLK), lambda i: (i, 0, 0))

    y2, r2, pool1 = pl.pallas_call(
        _tc2_body,
        grid=(N_BLKS,),
        in_specs=[
            acc_spec, deg_spec, blk_spec,
            _full_spec((1, D)), batch_spec,
            _full_spec((D, D)), _full_spec((D, D)),
        ],
        out_specs=[blk_spec, blk_spec, _full_spec((N_GRAPHS, D))],
        out_shape=[
            jax.ShapeDtypeStruct((ACC_ROWS, D), jnp.float32),
            jax.ShapeDtypeStruct((ACC_ROWS, D), jnp.float32),
            jax.ShapeDtypeStruct((N_GRAPHS, D), jnp.float32),
        ],
    )(acc1, degp, r1, b_rel1.reshape(1, D), batch3, W_rel2, W_root2)

    acc2 = _get_sc_agg()(y2, eidx)

    out = pl.pallas_call(
        _tc3_body,
        grid=(N_BLKS,),
        in_specs=[
            acc_spec, deg_spec, blk_spec,
            _full_spec((1, D)), batch_spec,
            _full_spec((N_GRAPHS, D)),
            _full_spec((2 * D, D)), _full_spec((1, D)),
            _full_spec((D, D // 2)), _full_spec((1, D // 2)),
            _full_spec((D // 2, 10)), _full_spec((1, 10)),
        ],
        out_specs=pl.BlockSpec((N_GRAPHS, 10), lambda i: (0, 0)),
        out_shape=jax.ShapeDtypeStruct((N_GRAPHS, 10), jnp.float32),
        scratch_shapes=[pltpu.VMEM((N_GRAPHS, D), jnp.float32)],
    )(acc2, degp, r2, b_rel2.reshape(1, D), batch3, pool1,
      W1, b1.reshape(1, D), W2, b2.reshape(1, D // 2), W3,
      b3.reshape(1, 10))

    return out


# speedup vs baseline: 1.9894x; 1.9894x over previous
import functools

import jax
import jax.numpy as jnp
from jax import lax
from jax.experimental import pallas as pl
from jax.experimental.pallas import tpu as pltpu
from jax.experimental.pallas import tpu_sc as plsc

N_NODES = 10000
D = 128
N_GRAPHS = 16
E = 320000

NC, NS = 2, 16
NW = NC * NS
CHUNK = 128
CPT0 = 100
CPT1 = 58
E_PAD = NS * (CPT0 + CPT1) * CHUNK
ACC_ROWS = 10240
ROWS_PER_TILE = ACC_ROWS // NS
ZROWS = 64

NODE_BLK = 1024
N_BLKS = ACC_ROWS // NODE_BLK


def _sc_agg_body(y_hbm, src_hbm, dst_hbm, acc_out, deg_out,
                 sidx_v, didx_v, rows_v, zbuf_v, deg_v, acc_sh, sem):
    cid = lax.axis_index("c")
    sid = lax.axis_index("s")
    wid = cid * NS + sid

    def _zrow(r, carry):
        for j in range(D // 16):
            zbuf_v[r, pl.ds(j * 16, 16)] = jnp.zeros((16,), jnp.float32)
        return carry
    lax.fori_loop(0, ZROWS, _zrow, 0)

    def _zdeg(i, carry):
        deg_v[pl.ds(i * 16, 16)] = jnp.zeros((16,), jnp.float32)
        return carry
    lax.fori_loop(0, ACC_ROWS // 16, _zdeg, 0)

    for j in range(ROWS_PER_TILE // ZROWS):
        pltpu.sync_copy(
            zbuf_v, acc_sh.at[pl.ds(sid * ROWS_PER_TILE + j * ZROWS, ZROWS)])
    plsc.subcore_barrier()

    ones = jnp.ones((16,), jnp.float32)
    ebase = jnp.where(cid == 0, sid * CPT0, NS * CPT0 + sid * CPT1) * CHUNK
    n_chunks = jnp.where(cid == 0, CPT0, CPT1)

    def _chunk(c, carry):
        base = ebase + c * CHUNK
        pltpu.sync_copy(src_hbm.at[pl.ds(base, CHUNK)], sidx_v)
        pltpu.sync_copy(dst_hbm.at[pl.ds(base, CHUNK)], didx_v)
        pltpu.async_copy(y_hbm.at[sidx_v], rows_v, sem).wait()
        pltpu.sync_copy(rows_v, acc_sh.at[didx_v], add=True)
        for j in range(CHUNK // 16):
            iv = didx_v[pl.ds(j * 16, 16)]
            plsc.addupdate_scatter(deg_v, [iv], ones)
        return carry
    lax.fori_loop(0, n_chunks, _chunk, 0)

    plsc.subcore_barrier()
    pltpu.sync_copy(
        acc_sh.at[pl.ds(sid * ROWS_PER_TILE, ROWS_PER_TILE)],
        acc_out.at[pl.ds(cid * ACC_ROWS + sid * ROWS_PER_TILE, ROWS_PER_TILE)])
    pltpu.sync_copy(deg_v, deg_out.at[wid])


@functools.cache
def _get_sc_agg():
    return functools.partial(
        pl.kernel,
        mesh=plsc.VectorSubcoreMesh(core_axis_name="c", subcore_axis_name="s"),
        compiler_params=pltpu.CompilerParams(needs_layout_passes=False),
        out_type=(
            jax.ShapeDtypeStruct((NC * ACC_ROWS, D), jnp.float32),
            jax.ShapeDtypeStruct((NW, ACC_ROWS), jnp.float32),
        ),
        scratch_types=[
            pltpu.VMEM((CHUNK,), jnp.int32),
            pltpu.VMEM((CHUNK,), jnp.int32),
            pltpu.VMEM((CHUNK, D), jnp.float32),
            pltpu.VMEM((ZROWS, D), jnp.float32),
            pltpu.VMEM((ACC_ROWS,), jnp.float32),
            pltpu.VMEM_SHARED((ACC_ROWS, D), jnp.float32),
            pltpu.SemaphoreType.DMA,
        ],
    )(_sc_agg_body)


def _tc1_body(x_ref, wa_ref, wb_ref, y_ref, r_ref):
    xb = x_ref[...]
    y_ref[...] = jnp.dot(xb, wa_ref[...], preferred_element_type=jnp.float32)
    r_ref[...] = jnp.dot(xb, wb_ref[...], preferred_element_type=jnp.float32)


def _conv_epilogue(acc_ref, deg_ref, r_ref, b_ref):
    deg = jnp.sum(deg_ref[...], axis=0)
    agg = acc_ref[0] + acc_ref[1]
    mean = agg / jnp.maximum(deg, 1.0)[:, None]
    return jnp.maximum(mean + b_ref[...] + r_ref[...], 0.0)


def _pool_contrib(batch_ref, h):
    b = batch_ref[0, 0, :]
    g = lax.broadcasted_iota(jnp.int32, (N_GRAPHS, NODE_BLK), 0)
    onehot = (g == b[None, :]).astype(jnp.float32)
    return jnp.dot(onehot, h, preferred_element_type=jnp.float32)


def _tc2_body(acc_ref, deg_ref, r_ref, b_ref, batch_ref, wa_ref, wb_ref,
              y2_ref, r2_ref, pool_ref):
    i = pl.program_id(0)
    h = _conv_epilogue(acc_ref, deg_ref, r_ref, b_ref)
    y2_ref[...] = jnp.dot(h, wa_ref[...], preferred_element_type=jnp.float32)
    r2_ref[...] = jnp.dot(h, wb_ref[...], preferred_element_type=jnp.float32)

    @pl.when(i == 0)
    def _():
        pool_ref[...] = jnp.zeros_like(pool_ref)
    pool_ref[...] += _pool_contrib(batch_ref, h)


def _tc3_body(acc_ref, deg_ref, r_ref, b_ref, batch_ref, pool1_ref,
              w1_ref, b1_ref, w2_ref, b2_ref, w3_ref, b3_ref,
              out_ref, pool2_ref):
    i = pl.program_id(0)
    h = _conv_epilogue(acc_ref, deg_ref, r_ref, b_ref)

    @pl.when(i == 0)
    def _():
        pool2_ref[...] = jnp.zeros_like(pool2_ref)
    pool2_ref[...] += _pool_contrib(batch_ref, h)

    @pl.when(i == N_BLKS - 1)
    def _():
        z = jnp.concatenate([pool1_ref[...], pool2_ref[...]], axis=-1)
        z = jnp.maximum(
            jnp.dot(z, w1_ref[...], preferred_element_type=jnp.float32)
            + b1_ref[...], 0.0)
        z = jnp.maximum(
            jnp.dot(z, w2_ref[...], preferred_element_type=jnp.float32)
            + b2_ref[...], 0.0)
        z = (jnp.dot(z, w3_ref[...], preferred_element_type=jnp.float32)
             + b3_ref[...])
        m = jnp.max(z, axis=-1, keepdims=True)
        lse = jnp.log(jnp.sum(jnp.exp(z - m), axis=-1, keepdims=True)) + m
        out_ref[...] = z - lse


def _full_spec(shape):
    return pl.BlockSpec(shape, lambda i: tuple(0 for _ in shape))


def kernel(x, edge_index, batch, W_rel1, b_rel1, W_root1, W_rel2, b_rel2,
           W_root2, W1, b1, W2, b2, W3, b3):
    src, dst = edge_index[0], edge_index[1]
    pad = E_PAD - E
    src_p = jnp.concatenate([src, jnp.zeros((pad,), jnp.int32)])
    pad_dst = N_NODES + jnp.arange(pad, dtype=jnp.int32) % (ACC_ROWS - N_NODES)
    dst_p = jnp.concatenate([dst, pad_dst])
    x_p = jnp.concatenate([x, jnp.zeros((ACC_ROWS - N_NODES, D), jnp.float32)])
    batch_p = jnp.concatenate(
        [batch, jnp.full((ACC_ROWS - N_NODES,), N_GRAPHS, jnp.int32)])
    batch3 = batch_p.reshape(N_BLKS, 1, NODE_BLK)

    y1, r1 = pl.pallas_call(
        _tc1_body,
        grid=(N_BLKS,),
        in_specs=[
            pl.BlockSpec((NODE_BLK, D), lambda i: (i, 0)),
            _full_spec((D, D)),
            _full_spec((D, D)),
        ],
        out_specs=[pl.BlockSpec((NODE_BLK, D), lambda i: (i, 0))] * 2,
        out_shape=[jax.ShapeDtypeStruct((ACC_ROWS, D), jnp.float32)] * 2,
    )(x_p, W_rel1, W_root1)

    acc1, degp = _get_sc_agg()(y1, src_p, dst_p)
    acc1 = acc1.reshape(NC, ACC_ROWS, D)

    blk_spec = pl.BlockSpec((NODE_BLK, D), lambda i: (i, 0))
    acc_spec = pl.BlockSpec((NC, NODE_BLK, D), lambda i: (0, i, 0))
    deg_spec = pl.BlockSpec((NW, NODE_BLK), lambda i: (0, i))
    batch_spec = pl.BlockSpec((1, 1, NODE_BLK), lambda i: (i, 0, 0))

    y2, r2, pool1 = pl.pallas_call(
        _tc2_body,
        grid=(N_BLKS,),
        in_specs=[
            acc_spec, deg_spec, blk_spec,
            _full_spec((1, D)), batch_spec,
            _full_spec((D, D)), _full_spec((D, D)),
        ],
        out_specs=[blk_spec, blk_spec, _full_spec((N_GRAPHS, D))],
        out_shape=[
            jax.ShapeDtypeStruct((ACC_ROWS, D), jnp.float32),
            jax.ShapeDtypeStruct((ACC_ROWS, D), jnp.float32),
            jax.ShapeDtypeStruct((N_GRAPHS, D), jnp.float32),
        ],
    )(acc1, degp, r1, b_rel1.reshape(1, D), batch3, W_rel2, W_root2)

    acc2, _deg2 = _get_sc_agg()(y2, src_p, dst_p)
    acc2 = acc2.reshape(NC, ACC_ROWS, D)

    out = pl.pallas_call(
        _tc3_body,
        grid=(N_BLKS,),
        in_specs=[
            acc_spec, deg_spec, blk_spec,
            _full_spec((1, D)), batch_spec,
            _full_spec((N_GRAPHS, D)),
            _full_spec((2 * D, D)), _full_spec((1, D)),
            _full_spec((D, D // 2)), _full_spec((1, D // 2)),
            _full_spec((D // 2, 10)), _full_spec((1, 10)),
        ],
        out_specs=pl.BlockSpec((N_GRAPHS, 10), lambda i: (0, 0)),
        out_shape=jax.ShapeDtypeStruct((N_GRAPHS, 10), jnp.float32),
        scratch_shapes=[pltpu.VMEM((N_GRAPHS, D), jnp.float32)],
    )(acc2, degp, r2, b_rel2.reshape(1, D), batch3, pool1,
      W1, b1.reshape(1, D), W2, b2.reshape(1, D // 2), W3,
      b3.reshape(1, 10))

    return out
